# baseline (speedup 1.0000x reference)
import functools

import numpy as np
import jax
import jax.numpy as jnp
from jax import lax
from jax.experimental import pallas as pl
from jax.experimental.pallas import tpu as pltpu
from jax.experimental.pallas import tpu_sc as plsc

N = 10000
E = 320000
IN_CH = 128
HID = 16
HEADS = 8
OUT_CH = 64

NC = 2
NS = 16
NW = NC * NS
LANES = 16

CHUNK = 64
E2 = E + N
PAIRB = NW * CHUNK * 2
NPAIR = (E2 + PAIRB - 1) // PAIRB
NCHUNK = 2 * NPAIR
EPW = NCHUNK * CHUNK
EP = EPW * NW

NR = 10016
STRIPE = NR // NS


def _vgather16(vec, idx):
    return lax.gather(
        vec, idx[:, None],
        lax.GatherDimensionNumbers(offset_dims=(), collapsed_slice_dims=(0,),
                                   start_index_map=(0,)),
        (1,), mode=lax.GatherScatterMode.PROMISE_IN_BOUNDS)


def _make_edge_kernel(kh, hid):
    hrw = kh * hid
    msgw = hrw + 16
    nvreg = hrw // LANES

    mesh = plsc.VectorSubcoreMesh(core_axis_name="c", subcore_axis_name="s")

    @functools.partial(
        pl.kernel,
        out_type=jax.ShapeDtypeStruct((NC, NR, msgw), jnp.float32),
        mesh=mesh,
        compiler_params=pltpu.CompilerParams(use_tc_tiling_on_sc=False),
        scratch_types=[
            pltpu.VMEM((CHUNK,), jnp.int32),
            pltpu.VMEM((CHUNK,), jnp.int32),
            pltpu.VMEM((CHUNK,), jnp.int32),
            pltpu.VMEM((CHUNK,), jnp.int32),
            pltpu.VMEM((CHUNK,), jnp.int32),
            pltpu.VMEM((CHUNK,), jnp.int32),
            pltpu.VMEM((CHUNK, 16), jnp.float32),
            pltpu.VMEM((CHUNK, 16), jnp.float32),
            pltpu.VMEM((CHUNK, 16), jnp.float32),
            pltpu.VMEM((CHUNK, 16), jnp.float32),
            pltpu.VMEM((CHUNK, hrw), jnp.float32),
            pltpu.VMEM((CHUNK, hrw), jnp.float32),
            pltpu.VMEM((CHUNK, msgw), jnp.float32),
            pltpu.VMEM((CHUNK, msgw), jnp.float32),
            pltpu.VMEM_SHARED((NR, msgw), jnp.float32),
        ] + [pltpu.SemaphoreType.DMA] * 12,
    )
    def edge_kernel(src_hbm, dst_hbm, ac_hbm, h_hbm, zeros_hbm, out_hbm,
                    src0, src1, dst0, dst1, dd0, dd1, sa0, sa1, da0, da1,
                    hr0, hr1, msg0, msg1, acc,
                    sS0, sS1, sDd0, sDd1, sSA0, sSA1, sDA0, sDA1,
                    sHR0, sHR1, sSC0, sSC1):
        srcb = (src0, src1)
        dstb = (dst0, dst1)
        ddb = (dd0, dd1)
        sab = (sa0, sa1)
        dab = (da0, da1)
        hrb = (hr0, hr1)
        msgb = (msg0, msg1)
        sS = (sS0, sS1)
        sDd = (sDd0, sDd1)
        sSA = (sSA0, sSA1)
        sDA = (sDA0, sDA1)
        sHR = (sHR0, sHR1)
        sSC = (sSC0, sSC1)

        cid = lax.axis_index("c")
        sid = lax.axis_index("s")
        wid = sid * NC + cid
        base = wid * EPW

        r0 = sid * STRIPE
        pltpu.sync_copy(zeros_hbm.at[pl.ds(r0, STRIPE)],
                        acc.at[pl.ds(r0, STRIPE)])
        plsc.subcore_barrier()

        lane = lax.iota(jnp.int32, 16)
        rot_idx = 8 + (lane & 7)
        head_mask = lane < kh

        def start_a(ci, b):
            off = base + ci * CHUNK
            pltpu.async_copy(src_hbm.at[pl.ds(off, CHUNK)], srcb[b], sS[b])
            pltpu.async_copy(dst_hbm.at[pl.ds(off, CHUNK)], dstb[b], sDd[b])

        def wait_a(b):
            pltpu.make_async_copy(src_hbm.at[pl.ds(0, CHUNK)], srcb[b],
                                  sS[b]).wait()
            pltpu.make_async_copy(dst_hbm.at[pl.ds(0, CHUNK)], dstb[b],
                                  sDd[b]).wait()

        def start_b(b):
            pltpu.async_copy(ac_hbm.at[srcb[b]], sab[b], sSA[b])
            pltpu.async_copy(ac_hbm.at[dstb[b]], dab[b], sDA[b])
            pltpu.async_copy(h_hbm.at[srcb[b]], hrb[b], sHR[b])

        def wait_b(b):
            pltpu.make_async_copy(ac_hbm.at[srcb[b]], sab[b], sSA[b]).wait()
            pltpu.make_async_copy(ac_hbm.at[dstb[b]], dab[b], sDA[b]).wait()
            pltpu.make_async_copy(h_hbm.at[srcb[b]], hrb[b], sHR[b]).wait()

        def start_d(b):
            pltpu.async_copy(msgb[b], acc.at[ddb[b]], sSC[b], add=True)

        def wait_d(b):
            pltpu.make_async_copy(msgb[b], acc.at[ddb[b]], sSC[b]).wait()

        def compute(b):
            sa_v, da_v, hr_v, msg_v = sab[b], dab[b], hrb[b], msgb[b]
            for q in range(CHUNK // 16):
                ddb[b][pl.ds(q * 16, 16)] = dstb[b][pl.ds(q * 16, 16)]

            @plsc.parallel_loop(0, CHUNK, 1, unroll=8)
            def edge_body(i):
                sa = sa_v[i, :]
                da = da_v[i, :]
                e = sa + _vgather16(da, rot_idx)
                w = jnp.exp(jnp.maximum(e, 0.2 * e))
                msg_v[i, pl.ds(hrw, 16)] = jnp.where(head_mask, w, 0.0)
                for v in range(nvreg):
                    h_lane = (v * LANES) // hid
                    wk = _vgather16(w, jnp.full((16,), h_lane, jnp.int32))
                    msg_v[i, pl.ds(v * LANES, LANES)] = (
                        wk * hr_v[i, pl.ds(v * LANES, LANES)])

        start_a(0, 0)
        start_a(1, 1)
        wait_a(0)
        start_b(0)

        def pair_body(g, _):
            ci0 = 2 * g
            wait_a(1)
            start_b(1)
            wait_b(0)

            @pl.when(g < NPAIR - 1)
            def _():
                start_a(ci0 + 2, 0)

            @pl.when(g > 0)
            def _():
                wait_d(0)

            compute(0)
            start_d(0)

            wait_b(1)

            @pl.when(g < NPAIR - 1)
            def _():
                start_a(ci0 + 3, 1)

            @pl.when(g > 0)
            def _():
                wait_d(1)

            compute(1)
            start_d(1)

            @pl.when(g < NPAIR - 1)
            def _():
                wait_a(0)
                start_b(0)

            return 0

        lax.fori_loop(0, NPAIR, pair_body, 0)
        wait_d(0)
        wait_d(1)
        plsc.subcore_barrier()
        pltpu.sync_copy(acc.at[pl.ds(r0, STRIPE)],
                        out_hbm.at[cid, pl.ds(r0, STRIPE)])

    return edge_kernel


_edge_l1 = _make_edge_kernel(HEADS, HID)
_edge_l2 = _make_edge_kernel(1, OUT_CH)


def _tc_pre_body(x_ref, w1_ref, am1_ref, h1_ref, a1_ref):
    h = jnp.dot(x_ref[...], w1_ref[...], preferred_element_type=jnp.float32)
    h1_ref[...] = h
    a1_ref[...] = jnp.dot(h, am1_ref[...], preferred_element_type=jnp.float32)


def _tc_pre(x, w1, am1):
    grid = (10,)
    return pl.pallas_call(
        _tc_pre_body,
        grid=grid,
        in_specs=[
            pl.BlockSpec((1000, IN_CH), lambda i: (i, 0)),
            pl.BlockSpec((IN_CH, IN_CH), lambda i: (0, 0)),
            pl.BlockSpec((IN_CH, 16), lambda i: (0, 0)),
        ],
        out_specs=[
            pl.BlockSpec((1000, IN_CH), lambda i: (i, 0)),
            pl.BlockSpec((1000, 16), lambda i: (i, 0)),
        ],
        out_shape=[
            jax.ShapeDtypeStruct((N, IN_CH), jnp.float32),
            jax.ShapeDtypeStruct((N, 16), jnp.float32),
        ],
    )(x, w1, am1)


def _tc_mid_body(p_ref, seln_ref, seld_ref, b1_ref, w2_ref, am2_ref,
                 h2_ref, a2_ref):
    rows = p_ref[0] + p_ref[1]
    numer = jnp.dot(rows, seln_ref[...], preferred_element_type=jnp.float32)
    denom = jnp.dot(rows, seld_ref[...], preferred_element_type=jnp.float32)
    out1 = numer / (denom + 1e-16) + b1_ref[...]
    x2 = jnp.where(out1 > 0, out1, jnp.exp(out1) - 1.0)
    h2 = jnp.dot(x2, w2_ref[...], preferred_element_type=jnp.float32)
    h2_ref[...] = h2
    a2_ref[...] = jnp.dot(h2, am2_ref[...], preferred_element_type=jnp.float32)


def _tc_mid(p, seln, seld, b1, w2, am2):
    grid = (4,)
    msgw = HEADS * HID + 16
    return pl.pallas_call(
        _tc_mid_body,
        grid=grid,
        in_specs=[
            pl.BlockSpec((2, 2504, msgw), lambda i: (0, i, 0)),
            pl.BlockSpec((msgw, IN_CH), lambda i: (0, 0)),
            pl.BlockSpec((msgw, IN_CH), lambda i: (0, 0)),
            pl.BlockSpec((1, IN_CH), lambda i: (0, 0)),
            pl.BlockSpec((IN_CH, OUT_CH), lambda i: (0, 0)),
            pl.BlockSpec((OUT_CH, 16), lambda i: (0, 0)),
        ],
        out_specs=[
            pl.BlockSpec((2504, OUT_CH), lambda i: (i, 0)),
            pl.BlockSpec((2504, 16), lambda i: (i, 0)),
        ],
        out_shape=[
            jax.ShapeDtypeStruct((NR, OUT_CH), jnp.float32),
            jax.ShapeDtypeStruct((NR, 16), jnp.float32),
        ],
    )(p, seln, seld, b1, w2, am2)


def _tc_post_body(p_ref, seln_ref, seld_ref, b2_ref, out_ref):
    rows = p_ref[0] + p_ref[1]
    numer = jnp.dot(rows, seln_ref[...], preferred_element_type=jnp.float32)
    denom = jnp.dot(rows, seld_ref[...], preferred_element_type=jnp.float32)
    out_ref[...] = numer / (denom + 1e-16) + b2_ref[...]


def _tc_post(p, seln, seld, b2):
    grid = (4,)
    msgw = OUT_CH + 16
    return pl.pallas_call(
        _tc_post_body,
        grid=grid,
        in_specs=[
            pl.BlockSpec((2, 2504, msgw), lambda i: (0, i, 0)),
            pl.BlockSpec((msgw, OUT_CH), lambda i: (0, 0)),
            pl.BlockSpec((msgw, OUT_CH), lambda i: (0, 0)),
            pl.BlockSpec((1, OUT_CH), lambda i: (0, 0)),
        ],
        out_specs=pl.BlockSpec((2504, OUT_CH), lambda i: (i, 0)),
        out_shape=jax.ShapeDtypeStruct((NR, OUT_CH), jnp.float32),
    )(p, seln, seld, b2)


_HEAD_OF = np.repeat(np.arange(HEADS), HID)

_MS1 = np.zeros((IN_CH, 16), np.float32)
_MS1[np.arange(IN_CH), _HEAD_OF] = 1.0
_MD1 = np.zeros((IN_CH, 16), np.float32)
_MD1[np.arange(IN_CH), _HEAD_OF + 8] = 1.0

_MS2 = np.zeros((OUT_CH, 16), np.float32)
_MS2[:, 0] = 1.0
_MD2 = np.zeros((OUT_CH, 16), np.float32)
_MD2[:, 8] = 1.0

_MSGW1 = HEADS * HID + 16
_SELN1 = np.zeros((_MSGW1, IN_CH), np.float32)
_SELN1[np.arange(IN_CH), np.arange(IN_CH)] = 1.0
_SELD1 = np.zeros((_MSGW1, IN_CH), np.float32)
_SELD1[IN_CH + _HEAD_OF, np.arange(IN_CH)] = 1.0

_MSGW2 = OUT_CH + 16
_SELN2 = np.zeros((_MSGW2, OUT_CH), np.float32)
_SELN2[np.arange(OUT_CH), np.arange(OUT_CH)] = 1.0
_SELD2 = np.zeros((_MSGW2, OUT_CH), np.float32)
_SELD2[OUT_CH, :] = 1.0


def kernel(x, edge_index, W1, a_src1, a_dst1, b1, W2, a_src2, a_dst2, b2):
    loops = jnp.arange(N, dtype=edge_index.dtype)
    src = jnp.concatenate([edge_index[0], loops])
    dst = jnp.concatenate([edge_index[1], loops])
    npad = EP - E2
    src_p = jnp.concatenate([src, jnp.zeros((npad,), src.dtype)]).astype(jnp.int32)
    pad_dst = N + (jnp.arange(npad, dtype=dst.dtype) % 16)
    dst_p = jnp.concatenate([dst, pad_dst]).astype(jnp.int32)

    as1 = a_src1.reshape(IN_CH, 1)
    ad1 = a_dst1.reshape(IN_CH, 1)
    am1 = jnp.asarray(_MS1) * as1 + jnp.asarray(_MD1) * ad1
    am2 = jnp.asarray(_MS2) * a_src2.reshape(OUT_CH, 1) + \
        jnp.asarray(_MD2) * a_dst2.reshape(OUT_CH, 1)

    zeros1 = jnp.zeros((NR, _MSGW1), jnp.float32)
    zeros2 = jnp.zeros((NR, _MSGW2), jnp.float32)

    h1, a1 = _tc_pre(x, W1, am1)
    part1 = _edge_l1(src_p, dst_p, a1, h1, zeros1)
    h2, a2 = _tc_mid(part1, jnp.asarray(_SELN1), jnp.asarray(_SELD1),
                     b1.reshape(1, IN_CH), W2, am2)
    part2 = _edge_l2(src_p, dst_p, a2, h2, zeros2)
    out = _tc_post(part2, jnp.asarray(_SELN2), jnp.asarray(_SELD2),
                   b2.reshape(1, OUT_CH))
    return out[:N]


# speedup vs baseline: 1.0346x; 1.0346x over previous
import functools

import numpy as np
import jax
import jax.numpy as jnp
from jax import lax
from jax.experimental import pallas as pl
from jax.experimental.pallas import tpu as pltpu
from jax.experimental.pallas import tpu_sc as plsc

N = 10000
E = 320000
IN_CH = 128
HID = 16
HEADS = 8
OUT_CH = 64

NC = 2
NS = 16
NW = NC * NS
LANES = 16

CHUNK = 64
E2 = E + N
PAIRB = NW * CHUNK * 2
NPAIR = (E2 + PAIRB - 1) // PAIRB
NCHUNK = 2 * NPAIR
EPW = NCHUNK * CHUNK
EP = EPW * NW

NR = 10016
STRIPE = NR // NS


def _vgather16(vec, idx):
    return lax.gather(
        vec, idx[:, None],
        lax.GatherDimensionNumbers(offset_dims=(), collapsed_slice_dims=(0,),
                                   start_index_map=(0,)),
        (1,), mode=lax.GatherScatterMode.PROMISE_IN_BOUNDS)


def _make_edge_kernel(kh, hid):
    hrw = kh * hid
    msgw = hrw + 16
    nvreg = hrw // LANES

    mesh = plsc.VectorSubcoreMesh(core_axis_name="c", subcore_axis_name="s")

    @functools.partial(
        pl.kernel,
        out_type=jax.ShapeDtypeStruct((NC, NR, msgw), jnp.float32),
        mesh=mesh,
        compiler_params=pltpu.CompilerParams(use_tc_tiling_on_sc=False),
        scratch_types=[
            pltpu.VMEM((CHUNK,), jnp.int32),
            pltpu.VMEM((CHUNK,), jnp.int32),
            pltpu.VMEM((CHUNK,), jnp.int32),
            pltpu.VMEM((CHUNK,), jnp.int32),
            pltpu.VMEM((CHUNK,), jnp.int32),
            pltpu.VMEM((CHUNK,), jnp.int32),
            pltpu.VMEM((CHUNK, 16), jnp.float32),
            pltpu.VMEM((CHUNK, 16), jnp.float32),
            pltpu.VMEM((CHUNK, 16), jnp.float32),
            pltpu.VMEM((CHUNK, 16), jnp.float32),
            pltpu.VMEM((CHUNK, hrw), jnp.float32),
            pltpu.VMEM((CHUNK, hrw), jnp.float32),
            pltpu.VMEM((CHUNK, msgw), jnp.float32),
            pltpu.VMEM((CHUNK, msgw), jnp.float32),
            pltpu.VMEM_SHARED((NR, msgw), jnp.float32),
        ] + [pltpu.SemaphoreType.DMA] * 12,
    )
    def edge_kernel(src_hbm, dst_hbm, as_hbm, ad_hbm, h_hbm, zeros_hbm, out_hbm,
                    src0, src1, dst0, dst1, dd0, dd1, sa0, sa1, da0, da1,
                    hr0, hr1, msg0, msg1, acc,
                    sS0, sS1, sDd0, sDd1, sSA0, sSA1, sDA0, sDA1,
                    sHR0, sHR1, sSC0, sSC1):
        srcb = (src0, src1)
        dstb = (dst0, dst1)
        ddb = (dd0, dd1)
        sab = (sa0, sa1)
        dab = (da0, da1)
        hrb = (hr0, hr1)
        msgb = (msg0, msg1)
        sS = (sS0, sS1)
        sDd = (sDd0, sDd1)
        sSA = (sSA0, sSA1)
        sDA = (sDA0, sDA1)
        sHR = (sHR0, sHR1)
        sSC = (sSC0, sSC1)

        cid = lax.axis_index("c")
        sid = lax.axis_index("s")
        wid = sid * NC + cid
        base = wid * EPW

        r0 = sid * STRIPE
        pltpu.sync_copy(zeros_hbm.at[pl.ds(r0, STRIPE)],
                        acc.at[pl.ds(r0, STRIPE)])
        plsc.subcore_barrier()

        lane = lax.iota(jnp.int32, 16)
        head_mask = lane < kh

        def start_a(ci, b):
            off = base + ci * CHUNK
            pltpu.async_copy(src_hbm.at[pl.ds(off, CHUNK)], srcb[b], sS[b])
            pltpu.async_copy(dst_hbm.at[pl.ds(off, CHUNK)], dstb[b], sDd[b])

        def wait_a(b):
            pltpu.make_async_copy(src_hbm.at[pl.ds(0, CHUNK)], srcb[b],
                                  sS[b]).wait()
            pltpu.make_async_copy(dst_hbm.at[pl.ds(0, CHUNK)], dstb[b],
                                  sDd[b]).wait()

        def start_b(b):
            pltpu.async_copy(as_hbm.at[srcb[b]], sab[b], sSA[b])
            pltpu.async_copy(ad_hbm.at[dstb[b]], dab[b], sDA[b])
            pltpu.async_copy(h_hbm.at[srcb[b]], hrb[b], sHR[b])

        def wait_b(b):
            pltpu.make_async_copy(as_hbm.at[srcb[b]], sab[b], sSA[b]).wait()
            pltpu.make_async_copy(ad_hbm.at[dstb[b]], dab[b], sDA[b]).wait()
            pltpu.make_async_copy(h_hbm.at[srcb[b]], hrb[b], sHR[b]).wait()

        def start_d(b):
            pltpu.async_copy(msgb[b], acc.at[ddb[b]], sSC[b], add=True)

        def wait_d(b):
            pltpu.make_async_copy(msgb[b], acc.at[ddb[b]], sSC[b]).wait()

        def compute(b):
            sa_v, da_v, hr_v, msg_v = sab[b], dab[b], hrb[b], msgb[b]
            for q in range(CHUNK // 16):
                ddb[b][pl.ds(q * 16, 16)] = dstb[b][pl.ds(q * 16, 16)]

            @plsc.parallel_loop(0, CHUNK, 1, unroll=8)
            def edge_body(i):
                sa = sa_v[i, :]
                da = da_v[i, :]
                e = sa + da
                w = jnp.exp(jnp.maximum(e, 0.2 * e))
                msg_v[i, pl.ds(hrw, 16)] = jnp.where(head_mask, w, 0.0)
                for v in range(nvreg):
                    h_lane = (v * LANES) // hid
                    wk = _vgather16(w, jnp.full((16,), h_lane, jnp.int32))
                    msg_v[i, pl.ds(v * LANES, LANES)] = (
                        wk * hr_v[i, pl.ds(v * LANES, LANES)])

        start_a(0, 0)
        start_a(1, 1)
        wait_a(0)
        start_b(0)

        def pair_body(g, _):
            ci0 = 2 * g
            wait_a(1)
            start_b(1)
            wait_b(0)

            @pl.when(g < NPAIR - 1)
            def _():
                start_a(ci0 + 2, 0)

            @pl.when(g > 0)
            def _():
                wait_d(0)

            compute(0)
            start_d(0)

            wait_b(1)

            @pl.when(g < NPAIR - 1)
            def _():
                start_a(ci0 + 3, 1)

            @pl.when(g > 0)
            def _():
                wait_d(1)

            compute(1)
            start_d(1)

            @pl.when(g < NPAIR - 1)
            def _():
                wait_a(0)
                start_b(0)

            return 0

        lax.fori_loop(0, NPAIR, pair_body, 0)
        wait_d(0)
        wait_d(1)
        plsc.subcore_barrier()
        pltpu.sync_copy(acc.at[pl.ds(r0, STRIPE)],
                        out_hbm.at[cid, pl.ds(r0, STRIPE)])

    return edge_kernel


_edge_l1 = _make_edge_kernel(HEADS, HID)
_edge_l2 = _make_edge_kernel(1, OUT_CH)


def _tc_pre_body(x_ref, w1_ref, ams_ref, amd_ref, h1_ref, as_ref, ad_ref):
    h = jnp.dot(x_ref[...], w1_ref[...], preferred_element_type=jnp.float32)
    h1_ref[...] = h
    as_ref[...] = jnp.dot(h, ams_ref[...], preferred_element_type=jnp.float32)
    ad_ref[...] = jnp.dot(h, amd_ref[...], preferred_element_type=jnp.float32)


def _tc_pre(x, w1, ams, amd):
    grid = (10,)
    return pl.pallas_call(
        _tc_pre_body,
        grid=grid,
        in_specs=[
            pl.BlockSpec((1000, IN_CH), lambda i: (i, 0)),
            pl.BlockSpec((IN_CH, IN_CH), lambda i: (0, 0)),
            pl.BlockSpec((IN_CH, 16), lambda i: (0, 0)),
            pl.BlockSpec((IN_CH, 16), lambda i: (0, 0)),
        ],
        out_specs=[
            pl.BlockSpec((1000, IN_CH), lambda i: (i, 0)),
            pl.BlockSpec((1000, 16), lambda i: (i, 0)),
            pl.BlockSpec((1000, 16), lambda i: (i, 0)),
        ],
        out_shape=[
            jax.ShapeDtypeStruct((N, IN_CH), jnp.float32),
            jax.ShapeDtypeStruct((N, 16), jnp.float32),
            jax.ShapeDtypeStruct((N, 16), jnp.float32),
        ],
    )(x, w1, ams, amd)


def _tc_mid_body(p_ref, seln_ref, seld_ref, b1_ref, w2_ref, ams2_ref,
                 amd2_ref, h2_ref, as2_ref, ad2_ref):
    rows = p_ref[0] + p_ref[1]
    numer = jnp.dot(rows, seln_ref[...], preferred_element_type=jnp.float32)
    denom = jnp.dot(rows, seld_ref[...], preferred_element_type=jnp.float32)
    out1 = numer / (denom + 1e-16) + b1_ref[...]
    x2 = jnp.where(out1 > 0, out1, jnp.exp(out1) - 1.0)
    h2 = jnp.dot(x2, w2_ref[...], preferred_element_type=jnp.float32)
    h2_ref[...] = h2
    as2_ref[...] = jnp.dot(h2, ams2_ref[...], preferred_element_type=jnp.float32)
    ad2_ref[...] = jnp.dot(h2, amd2_ref[...], preferred_element_type=jnp.float32)


def _tc_mid(p, seln, seld, b1, w2, ams2, amd2):
    grid = (4,)
    msgw = HEADS * HID + 16
    return pl.pallas_call(
        _tc_mid_body,
        grid=grid,
        in_specs=[
            pl.BlockSpec((2, 2504, msgw), lambda i: (0, i, 0)),
            pl.BlockSpec((msgw, IN_CH), lambda i: (0, 0)),
            pl.BlockSpec((msgw, IN_CH), lambda i: (0, 0)),
            pl.BlockSpec((1, IN_CH), lambda i: (0, 0)),
            pl.BlockSpec((IN_CH, OUT_CH), lambda i: (0, 0)),
            pl.BlockSpec((OUT_CH, 16), lambda i: (0, 0)),
            pl.BlockSpec((OUT_CH, 16), lambda i: (0, 0)),
        ],
        out_specs=[
            pl.BlockSpec((2504, OUT_CH), lambda i: (i, 0)),
            pl.BlockSpec((2504, 16), lambda i: (i, 0)),
            pl.BlockSpec((2504, 16), lambda i: (i, 0)),
        ],
        out_shape=[
            jax.ShapeDtypeStruct((NR, OUT_CH), jnp.float32),
            jax.ShapeDtypeStruct((NR, 16), jnp.float32),
            jax.ShapeDtypeStruct((NR, 16), jnp.float32),
        ],
    )(p, seln, seld, b1, w2, ams2, amd2)


def _tc_post_body(p_ref, seln_ref, seld_ref, b2_ref, out_ref):
    rows = p_ref[0] + p_ref[1]
    numer = jnp.dot(rows, seln_ref[...], preferred_element_type=jnp.float32)
    denom = jnp.dot(rows, seld_ref[...], preferred_element_type=jnp.float32)
    out_ref[...] = numer / (denom + 1e-16) + b2_ref[...]


def _tc_post(p, seln, seld, b2):
    grid = (4,)
    msgw = OUT_CH + 16
    return pl.pallas_call(
        _tc_post_body,
        grid=grid,
        in_specs=[
            pl.BlockSpec((2, 2504, msgw), lambda i: (0, i, 0)),
            pl.BlockSpec((msgw, OUT_CH), lambda i: (0, 0)),
            pl.BlockSpec((msgw, OUT_CH), lambda i: (0, 0)),
            pl.BlockSpec((1, OUT_CH), lambda i: (0, 0)),
        ],
        out_specs=pl.BlockSpec((2504, OUT_CH), lambda i: (i, 0)),
        out_shape=jax.ShapeDtypeStruct((NR, OUT_CH), jnp.float32),
    )(p, seln, seld, b2)


_HEAD_OF = np.repeat(np.arange(HEADS), HID)

_MS1 = np.zeros((IN_CH, 16), np.float32)
_MS1[np.arange(IN_CH), _HEAD_OF] = 1.0
_MD1 = np.zeros((IN_CH, 16), np.float32)
_MD1[np.arange(IN_CH), _HEAD_OF] = 1.0

_MS2 = np.zeros((OUT_CH, 16), np.float32)
_MS2[:, 0] = 1.0
_MD2 = np.zeros((OUT_CH, 16), np.float32)
_MD2[:, 0] = 1.0

_MSGW1 = HEADS * HID + 16
_SELN1 = np.zeros((_MSGW1, IN_CH), np.float32)
_SELN1[np.arange(IN_CH), np.arange(IN_CH)] = 1.0
_SELD1 = np.zeros((_MSGW1, IN_CH), np.float32)
_SELD1[IN_CH + _HEAD_OF, np.arange(IN_CH)] = 1.0

_MSGW2 = OUT_CH + 16
_SELN2 = np.zeros((_MSGW2, OUT_CH), np.float32)
_SELN2[np.arange(OUT_CH), np.arange(OUT_CH)] = 1.0
_SELD2 = np.zeros((_MSGW2, OUT_CH), np.float32)
_SELD2[OUT_CH, :] = 1.0


def kernel(x, edge_index, W1, a_src1, a_dst1, b1, W2, a_src2, a_dst2, b2):
    loops = jnp.arange(N, dtype=edge_index.dtype)
    src = jnp.concatenate([edge_index[0], loops])
    dst = jnp.concatenate([edge_index[1], loops])
    npad = EP - E2
    src_p = jnp.concatenate([src, jnp.zeros((npad,), src.dtype)]).astype(jnp.int32)
    pad_dst = N + (jnp.arange(npad, dtype=dst.dtype) % 16)
    dst_p = jnp.concatenate([dst, pad_dst]).astype(jnp.int32)

    ams1 = jnp.asarray(_MS1) * a_src1.reshape(IN_CH, 1)
    amd1 = jnp.asarray(_MD1) * a_dst1.reshape(IN_CH, 1)
    ams2 = jnp.asarray(_MS2) * a_src2.reshape(OUT_CH, 1)
    amd2 = jnp.asarray(_MD2) * a_dst2.reshape(OUT_CH, 1)

    zeros1 = jnp.zeros((NR, _MSGW1), jnp.float32)
    zeros2 = jnp.zeros((NR, _MSGW2), jnp.float32)

    h1, a1s, a1d = _tc_pre(x, W1, ams1, amd1)
    part1 = _edge_l1(src_p, dst_p, a1s, a1d, h1, zeros1)
    h2, a2s, a2d = _tc_mid(part1, jnp.asarray(_SELN1), jnp.asarray(_SELD1),
                           b1.reshape(1, IN_CH), W2, ams2, amd2)
    part2 = _edge_l2(src_p, dst_p, a2s, a2d, h2, zeros2)
    out = _tc_post(part2, jnp.asarray(_SELN2), jnp.asarray(_SELD2),
                   b2.reshape(1, OUT_CH))
    return out[:N]
